# baseline (speedup 1.0000x reference)
import functools

import jax
import jax.numpy as jnp
from jax import lax
from jax.experimental import pallas as pl
from jax.experimental.pallas import tpu as pltpu
from jax.experimental.pallas import tpu_sc as plsc

B = 4096
L = 200
D = 64
T = B * L

NW = 32
PER_W = T // NW
C = 128
NSLOT = 4
MACRO = PER_W // (C * NSLOT)

_mesh = plsc.VectorSubcoreMesh(core_axis_name="c", subcore_axis_name="s")


@functools.partial(
    pl.kernel,
    mesh=_mesh,
    compiler_params=pltpu.CompilerParams(use_tc_tiling_on_sc=False),
    out_type=jax.ShapeDtypeStruct((T, D), jnp.float32),
    scratch_types=[
        pltpu.VMEM((NSLOT, C), jnp.int32),
        pltpu.VMEM((NSLOT, C), jnp.int32),
        pltpu.VMEM((NSLOT, C), jnp.int32),
        pltpu.VMEM((NSLOT, C, D), jnp.float32),
        pltpu.VMEM((NSLOT, C, D), jnp.float32),
        pltpu.VMEM((NSLOT, C, D), jnp.float32),
    ]
    + [pltpu.SemaphoreType.DMA] * (3 * NSLOT),
)
def _emb_lookup(tok_hbm, row_hbm, col_hbm, ttab, rtab, ctab, out_hbm,
                idx_t, idx_r, idx_c, buf_t, buf_r, buf_c, *sems):
    s_idx = sems[0:NSLOT]
    s_gat = sems[NSLOT:2 * NSLOT]
    s_out = sems[2 * NSLOT:3 * NSLOT]
    wid = lax.axis_index("s") * 2 + lax.axis_index("c")
    base0 = wid * PER_W

    def issue_idx(j, chunk):
        src = pl.ds(base0 + chunk * C, C)
        pltpu.async_copy(tok_hbm.at[src], idx_t.at[j], s_idx[j])
        pltpu.async_copy(row_hbm.at[src], idx_r.at[j], s_idx[j])
        pltpu.async_copy(col_hbm.at[src], idx_c.at[j], s_idx[j])

    def wait_idx(j):
        pltpu.make_async_copy(tok_hbm.at[pl.ds(0, C)], idx_t.at[j], s_idx[j]).wait()
        pltpu.make_async_copy(row_hbm.at[pl.ds(0, C)], idx_r.at[j], s_idx[j]).wait()
        pltpu.make_async_copy(col_hbm.at[pl.ds(0, C)], idx_c.at[j], s_idx[j]).wait()

    def issue_gathers(j):
        pltpu.async_copy(ttab.at[idx_t.at[j]], buf_t.at[j], s_gat[j])
        pltpu.async_copy(rtab.at[idx_r.at[j]], buf_r.at[j], s_gat[j])
        pltpu.async_copy(ctab.at[idx_c.at[j]], buf_c.at[j], s_gat[j])

    def wait_gathers(j):
        pltpu.make_async_copy(ttab.at[idx_t.at[j]], buf_t.at[j], s_gat[j]).wait()
        pltpu.make_async_copy(rtab.at[idx_r.at[j]], buf_r.at[j], s_gat[j]).wait()
        pltpu.make_async_copy(ctab.at[idx_c.at[j]], buf_c.at[j], s_gat[j]).wait()

    def issue_out(j, chunk):
        dst = pl.ds(base0 + chunk * C, C)
        pltpu.async_copy(buf_t.at[j], out_hbm.at[dst], s_out[j])

    def wait_out(j):
        pltpu.make_async_copy(buf_t.at[j], out_hbm.at[pl.ds(0, C)], s_out[j]).wait()

    for j in range(NSLOT):
        issue_idx(j, j)
    for j in range(NSLOT):
        wait_idx(j)
        issue_gathers(j)

    def macro_body(m, carry):
        chunk0 = m * NSLOT
        for j in range(NSLOT):
            wait_gathers(j)

            def row_body(i, c2):
                for dd in range(D // 16):
                    sl = pl.ds(dd * 16, 16)
                    v = buf_r[j, i, sl] + buf_c[j, i, sl]
                    plsc.addupdate(buf_t.at[j, i, sl], v)
                return c2

            lax.fori_loop(0, C, row_body, 0, unroll=8)
            issue_out(j, chunk0 + j)
            @pl.when(m < MACRO - 1)
            def _():
                issue_idx(j, chunk0 + NSLOT + j)

            if j >= 2:
                @pl.when(m < MACRO - 1)
                def _():
                    wait_idx(j - 2)
                    wait_out(j - 2)
                    issue_gathers(j - 2)

        @pl.when(m < MACRO - 1)
        def _():
            for k in (NSLOT - 2, NSLOT - 1):
                wait_idx(k)
                wait_out(k)
                issue_gathers(k)

        return carry

    lax.fori_loop(0, MACRO, macro_body, 0)
    for j in range(NSLOT):
        wait_out(j)


def kernel(tokens, row_indices, col_indices, token_table, row_table, col_table):
    tok = tokens.reshape(T).astype(jnp.int32)
    ri = row_indices.reshape(T).astype(jnp.int32)
    ci = col_indices.reshape(T).astype(jnp.int32)
    out = _emb_lookup(tok, ri, ci, token_table, row_table, col_table)
    return out.reshape(B, L, D)
